# baseline (speedup 1.0000x reference)
import functools

import jax
import jax.numpy as jnp
import numpy as np
from jax import lax
from jax.experimental import pallas as pl
from jax.experimental.pallas import tpu as pltpu
from jax.experimental.pallas import tpu_sc as plsc

_GATE_RATE = 0.3
_TEMPERATURE = 0.1

_NC = 2
_NS = 16
_NW = _NC * _NS
_LANES = 16


def _sc_gather(flat_logits, flat_actions, vocab):
    n = flat_actions.shape[0]
    per = n // _NW
    mesh = plsc.VectorSubcoreMesh(
        core_axis_name="c", subcore_axis_name="s",
        num_cores=_NC, num_subcores=_NS)

    @functools.partial(
        pl.kernel,
        out_type=jax.ShapeDtypeStruct((n,), jnp.float32),
        mesh=mesh,
        scratch_types=[
            pltpu.VMEM((per,), jnp.int32),
            pltpu.VMEM((per,), jnp.int32),
            pltpu.VMEM((per,), jnp.float32),
            pltpu.SemaphoreType.DMA,
        ],
    )
    def gather_kernel(flat_hbm, act_hbm, out_hbm, act_v, idx_v, val_v, sem):
        wid = lax.axis_index("s") * _NC + lax.axis_index("c")
        base = wid * per
        pltpu.sync_copy(act_hbm.at[pl.ds(base, per)], act_v)
        row = base + lax.iota(jnp.int32, _LANES)
        idx_v[...] = row * vocab + act_v[...]
        pltpu.async_copy(flat_hbm.at[idx_v], val_v, sem).wait()
        pltpu.sync_copy(val_v, out_hbm.at[pl.ds(base, per)])

    return gather_kernel(flat_logits, flat_actions)


def _tc_logz(x2d, chunk=2048):
    n, v = x2d.shape
    grid = pl.cdiv(v, chunk)

    def body(x_ref, o_ref, m_acc, s_acc):
        j = pl.program_id(0)

        @pl.when(j == 0)
        def _init():
            m_acc[...] = jnp.full(m_acc.shape, -jnp.inf, jnp.float32)
            s_acc[...] = jnp.zeros(s_acc.shape, jnp.float32)

        x = x_ref[...]
        col = j * chunk + lax.broadcasted_iota(jnp.int32, x.shape, 1)
        x = jnp.where(col < v, x, -jnp.inf)
        m_c = jnp.max(x, axis=1, keepdims=True)
        s_c = jnp.sum(jnp.exp(x - m_c), axis=1, keepdims=True)
        m_old = m_acc[...]
        m_new = jnp.maximum(m_old, m_c)
        s_acc[...] = s_acc[...] * jnp.exp(m_old - m_new) + s_c * jnp.exp(m_c - m_new)
        m_acc[...] = m_new

        @pl.when(j == grid - 1)
        def _fin():
            o_ref[...] = m_acc[...] + jnp.log(s_acc[...])

    return pl.pallas_call(
        body,
        grid=(grid,),
        in_specs=[pl.BlockSpec((n, chunk), lambda j: (0, j))],
        out_specs=pl.BlockSpec((n, 1), lambda j: (0, 0)),
        out_shape=jax.ShapeDtypeStruct((n, 1), jnp.float32),
        scratch_shapes=[
            pltpu.VMEM((n, 1), jnp.float32),
            pltpu.VMEM((n, 1), jnp.float32),
        ],
    )(x2d)


def _tc_tail(g2, z2, gt, zt, advt, maskt, u):
    b, t = g2.shape
    loc = np.float32(1.0 - _GATE_RATE) * np.float32(b - 1)
    lo = int(np.floor(loc))
    hi = int(np.ceil(loc))
    if lo == hi:
        w_lo, w_hi = np.float32(1.0), np.float32(0.0)
    else:
        w_lo = np.float32(hi) - loc
        w_hi = loc - np.float32(lo)

    def body(g_ref, z_ref, gt_ref, zt_ref, advt_ref, maskt_ref, u_ref,
             alp_ref, gw_ref, gp_ref, dl_ref, price_ref, rate_ref, loss_ref):
        alp_ref[...] = g_ref[...] - z_ref[...]
        alpt = gt_ref[...] - zt_ref[...]
        mt = maskt_ref[...]
        denom = jnp.clip(jnp.sum(mt, axis=0, keepdims=True), 1.0, None)
        mlp = jnp.sum(alpt * mt, axis=0, keepdims=True) / denom
        madv = jnp.sum(advt_ref[...] * mt, axis=0, keepdims=True) / denom
        dl = madv * (-mlp)
        dl_ref[...] = dl
        x = dl * jnp.ones((b, 1), jnp.float32)
        ri = lax.broadcasted_iota(jnp.int32, (b, b), 0)
        ci = lax.broadcasted_iota(jnp.int32, (b, b), 1)
        diag = jnp.where(ri == ci, x, 0.0)
        colv = jnp.sum(diag, axis=1, keepdims=True)
        a = colv * jnp.ones((1, b), jnp.float32)
        before = (x < a) | ((x == a) & (ci < ri))
        rank = jnp.sum(before.astype(jnp.float32), axis=1, keepdims=True)
        s_lo = jnp.sum(jnp.where(rank == lo, colv, 0.0), axis=0, keepdims=True)
        s_hi = jnp.sum(jnp.where(rank == hi, colv, 0.0), axis=0, keepdims=True)
        price = jnp.sum(s_lo * w_lo + s_hi * w_hi, axis=1, keepdims=True)
        price_ref[...] = price
        gl = (dl - price) / jnp.float32(_TEMPERATURE)
        gp = jax.nn.sigmoid(gl)
        gp_ref[...] = gp
        samples = (u_ref[...] < gp).astype(jnp.float32)
        gw = (samples + gp) - gp
        gw_ref[...] = gw
        rate_ref[...] = jnp.sum(samples, axis=1, keepdims=True) / np.float32(b)
        num = -jnp.sum(((gw * advt_ref[...]) * alpt) * mt)
        den = jnp.clip(jnp.sum(mt), 1.0, None)
        loss_ref[...] = jnp.reshape(num / den, (1, 1))

    outs = pl.pallas_call(
        body,
        out_shape=(
            jax.ShapeDtypeStruct((b, t), jnp.float32),
            jax.ShapeDtypeStruct((1, b), jnp.float32),
            jax.ShapeDtypeStruct((1, b), jnp.float32),
            jax.ShapeDtypeStruct((1, b), jnp.float32),
            jax.ShapeDtypeStruct((1, 1), jnp.float32),
            jax.ShapeDtypeStruct((1, 1), jnp.float32),
            jax.ShapeDtypeStruct((1, 1), jnp.float32),
        ),
    )(g2, z2, gt, zt, advt, maskt, u)
    return outs


def kernel(logits, actions, advantages, attention_mask):
    b, t, v = logits.shape
    n = b * t
    x2d = logits.reshape(n, v)
    flat_logits = logits.reshape(n * v)
    flat_actions = actions.reshape(n).astype(jnp.int32)

    gathered = _sc_gather(flat_logits, flat_actions, v)
    logz = _tc_logz(x2d)

    g2 = gathered.reshape(b, t)
    z2 = logz.reshape(b, t)
    mask = attention_mask.astype(jnp.float32)
    u = jax.random.uniform(jax.random.key(42), (b,), jnp.float32).reshape(1, b)

    alp, gw, gp, dl, price, rate, loss = _tc_tail(
        g2, z2, g2.T, z2.T, advantages.astype(jnp.float32).T, mask.T, u)

    return (
        gw.reshape(b),
        gp.reshape(b),
        dl.reshape(b),
        price.reshape(()),
        rate.reshape(()),
        loss.reshape(()),
        alp,
    )


# speedup vs baseline: 3.7900x; 3.7900x over previous
import functools

import jax
import jax.numpy as jnp
import numpy as np
from jax import lax
from jax.experimental import pallas as pl
from jax.experimental.pallas import tpu as pltpu
from jax.experimental.pallas import tpu_sc as plsc

_GATE_RATE = 0.3
_TEMPERATURE = 0.1

_NC = 2
_NS = 16
_NW = _NC * _NS
_LANES = 16


def _sc_gather(flat_logits, flat_actions, vocab):
    n = flat_actions.shape[0]
    per = n // _NW
    mesh = plsc.VectorSubcoreMesh(
        core_axis_name="c", subcore_axis_name="s",
        num_cores=_NC, num_subcores=_NS)

    @functools.partial(
        pl.kernel,
        out_type=jax.ShapeDtypeStruct((n,), jnp.float32),
        mesh=mesh,
        scratch_types=[
            pltpu.VMEM((per,), jnp.int32),
            pltpu.VMEM((per,), jnp.int32),
            pltpu.VMEM((per,), jnp.float32),
            pltpu.SemaphoreType.DMA,
        ],
    )
    def gather_kernel(flat_hbm, act_hbm, out_hbm, act_v, idx_v, val_v, sem):
        wid = lax.axis_index("s") * _NC + lax.axis_index("c")
        base = wid * per
        pltpu.sync_copy(act_hbm.at[pl.ds(base, per)], act_v)
        row = base + lax.iota(jnp.int32, _LANES)
        idx_v[...] = row * vocab + act_v[...]
        pltpu.async_copy(flat_hbm.at[idx_v], val_v, sem).wait()
        pltpu.sync_copy(val_v, out_hbm.at[pl.ds(base, per)])

    return gather_kernel(flat_logits, flat_actions)


def _tc_logz_gather(x2d, acts, chunk=2048):
    n, v = x2d.shape
    grid = pl.cdiv(v, chunk)

    def body(x_ref, a_ref, o_ref, g_ref, m_acc, s_acc, g_acc):
        j = pl.program_id(0)

        @pl.when(j == 0)
        def _init():
            m_acc[...] = jnp.full(m_acc.shape, -jnp.inf, jnp.float32)
            s_acc[...] = jnp.zeros(s_acc.shape, jnp.float32)
            g_acc[...] = jnp.zeros(g_acc.shape, jnp.float32)

        x = x_ref[...]
        loc = lax.broadcasted_iota(jnp.int32, x.shape, 1)
        a_adj = a_ref[...] - j * chunk
        g_acc[...] += jnp.sum(jnp.where(loc == a_adj, x, 0.0), axis=1,
                              keepdims=True)

        def merge(xm):
            m_c = jnp.max(xm, axis=1, keepdims=True)
            s_c = jnp.sum(jnp.exp(xm - m_c), axis=1, keepdims=True)
            m_old = m_acc[...]
            m_new = jnp.maximum(m_old, m_c)
            s_acc[...] = (s_acc[...] * jnp.exp(m_old - m_new)
                          + s_c * jnp.exp(m_c - m_new))
            m_acc[...] = m_new

        @pl.when(j < grid - 1)
        def _full():
            merge(x)

        @pl.when(j == grid - 1)
        def _last():
            merge(jnp.where(loc < v - j * chunk, x, -jnp.inf))
            o_ref[...] = m_acc[...] + jnp.log(s_acc[...])
            g_ref[...] = g_acc[...]

    return pl.pallas_call(
        body,
        grid=(grid,),
        in_specs=[
            pl.BlockSpec((n, chunk), lambda j: (0, j)),
            pl.BlockSpec((n, 1), lambda j: (0, 0)),
        ],
        out_specs=(
            pl.BlockSpec((n, 1), lambda j: (0, 0)),
            pl.BlockSpec((n, 1), lambda j: (0, 0)),
        ),
        out_shape=(
            jax.ShapeDtypeStruct((n, 1), jnp.float32),
            jax.ShapeDtypeStruct((n, 1), jnp.float32),
        ),
        scratch_shapes=[
            pltpu.VMEM((n, 1), jnp.float32),
            pltpu.VMEM((n, 1), jnp.float32),
            pltpu.VMEM((n, 1), jnp.float32),
        ],
    )(x2d, acts)


def _tc_tail(g2, z2, gt, zt, advt, maskt, u):
    b, t = g2.shape
    loc = np.float32(1.0 - _GATE_RATE) * np.float32(b - 1)
    lo = int(np.floor(loc))
    hi = int(np.ceil(loc))
    if lo == hi:
        w_lo, w_hi = np.float32(1.0), np.float32(0.0)
    else:
        w_lo = np.float32(hi) - loc
        w_hi = loc - np.float32(lo)

    def body(g_ref, z_ref, gt_ref, zt_ref, advt_ref, maskt_ref, u_ref,
             alp_ref, gw_ref, gp_ref, dl_ref, price_ref, rate_ref, loss_ref):
        alp_ref[...] = g_ref[...] - z_ref[...]
        alpt = gt_ref[...] - zt_ref[...]
        mt = maskt_ref[...]
        denom = jnp.clip(jnp.sum(mt, axis=0, keepdims=True), 1.0, None)
        mlp = jnp.sum(alpt * mt, axis=0, keepdims=True) / denom
        madv = jnp.sum(advt_ref[...] * mt, axis=0, keepdims=True) / denom
        dl = madv * (-mlp)
        dl_ref[...] = dl
        x = dl * jnp.ones((b, 1), jnp.float32)
        ri = lax.broadcasted_iota(jnp.int32, (b, b), 0)
        ci = lax.broadcasted_iota(jnp.int32, (b, b), 1)
        diag = jnp.where(ri == ci, x, 0.0)
        colv = jnp.sum(diag, axis=1, keepdims=True)
        a = colv * jnp.ones((1, b), jnp.float32)
        before = (x < a) | ((x == a) & (ci < ri))
        rank = jnp.sum(before.astype(jnp.float32), axis=1, keepdims=True)
        s_lo = jnp.sum(jnp.where(rank == lo, colv, 0.0), axis=0, keepdims=True)
        s_hi = jnp.sum(jnp.where(rank == hi, colv, 0.0), axis=0, keepdims=True)
        price = jnp.sum(s_lo * w_lo + s_hi * w_hi, axis=1, keepdims=True)
        price_ref[...] = price
        gl = (dl - price) / jnp.float32(_TEMPERATURE)
        gp = jax.nn.sigmoid(gl)
        gp_ref[...] = gp
        samples = (u_ref[...] < gp).astype(jnp.float32)
        gw = (samples + gp) - gp
        gw_ref[...] = gw
        rate_ref[...] = jnp.sum(samples, axis=1, keepdims=True) / np.float32(b)
        num = -jnp.sum(((gw * advt_ref[...]) * alpt) * mt)
        den = jnp.clip(jnp.sum(mt), 1.0, None)
        loss_ref[...] = jnp.reshape(num / den, (1, 1))

    outs = pl.pallas_call(
        body,
        out_shape=(
            jax.ShapeDtypeStruct((b, t), jnp.float32),
            jax.ShapeDtypeStruct((1, b), jnp.float32),
            jax.ShapeDtypeStruct((1, b), jnp.float32),
            jax.ShapeDtypeStruct((1, b), jnp.float32),
            jax.ShapeDtypeStruct((1, 1), jnp.float32),
            jax.ShapeDtypeStruct((1, 1), jnp.float32),
            jax.ShapeDtypeStruct((1, 1), jnp.float32),
        ),
    )(g2, z2, gt, zt, advt, maskt, u)
    return outs


def kernel(logits, actions, advantages, attention_mask):
    b, t, v = logits.shape
    n = b * t
    x2d = logits.reshape(n, v)
    acts = actions.reshape(n, 1).astype(jnp.int32)

    logz, gathered = _tc_logz_gather(x2d, acts)

    g2 = gathered.reshape(b, t)
    z2 = logz.reshape(b, t)
    mask = attention_mask.astype(jnp.float32)
    u = jax.random.uniform(jax.random.key(42), (b,), jnp.float32).reshape(1, b)

    alp, gw, gp, dl, price, rate, loss = _tc_tail(
        g2, z2, g2.T, z2.T, advantages.astype(jnp.float32).T, mask.T, u)

    return (
        gw.reshape(b),
        gp.reshape(b),
        dl.reshape(b),
        price.reshape(()),
        rate.reshape(()),
        loss.reshape(()),
        alp,
    )


# speedup vs baseline: 4.2117x; 1.1113x over previous
import functools

import jax
import jax.numpy as jnp
import numpy as np
from jax import lax
from jax.experimental import pallas as pl
from jax.experimental.pallas import tpu as pltpu
from jax.experimental.pallas import tpu_sc as plsc

_GATE_RATE = 0.3
_TEMPERATURE = 0.1

_NC = 2
_NS = 16
_NW = _NC * _NS
_LANES = 16


def _sc_gather(flat_logits, flat_actions, vocab):
    n = flat_actions.shape[0]
    per = n // _NW
    mesh = plsc.VectorSubcoreMesh(
        core_axis_name="c", subcore_axis_name="s",
        num_cores=_NC, num_subcores=_NS)

    @functools.partial(
        pl.kernel,
        out_type=jax.ShapeDtypeStruct((n,), jnp.float32),
        mesh=mesh,
        scratch_types=[
            pltpu.VMEM((per,), jnp.int32),
            pltpu.VMEM((per,), jnp.int32),
            pltpu.VMEM((per,), jnp.float32),
            pltpu.SemaphoreType.DMA,
        ],
    )
    def gather_kernel(flat_hbm, act_hbm, out_hbm, act_v, idx_v, val_v, sem):
        wid = lax.axis_index("s") * _NC + lax.axis_index("c")
        base = wid * per
        pltpu.sync_copy(act_hbm.at[pl.ds(base, per)], act_v)
        row = base + lax.iota(jnp.int32, _LANES)
        idx_v[...] = row * vocab + act_v[...]
        pltpu.async_copy(flat_hbm.at[idx_v], val_v, sem).wait()
        pltpu.sync_copy(val_v, out_hbm.at[pl.ds(base, per)])

    return gather_kernel(flat_logits, flat_actions)


def _tc_logz_gather(x2d, acts, chunk=4096):
    n, v = x2d.shape
    grid = pl.cdiv(v, chunk)

    def body(x_ref, a_ref, o_ref, g_ref, m_acc, s_acc, g_acc):
        j = pl.program_id(0)

        @pl.when(j == 0)
        def _init():
            m_acc[...] = jnp.full(m_acc.shape, -jnp.inf, jnp.float32)
            s_acc[...] = jnp.zeros(s_acc.shape, jnp.float32)
            g_acc[...] = jnp.zeros(g_acc.shape, jnp.float32)

        x = x_ref[...]
        loc = lax.broadcasted_iota(jnp.int32, x.shape, 1)
        a_adj = a_ref[...] - j * chunk
        g_acc[...] += jnp.sum(jnp.where(loc == a_adj, x, 0.0), axis=1,
                              keepdims=True)

        def merge(xm):
            m_c = jnp.max(xm, axis=1, keepdims=True)
            s_c = jnp.sum(jnp.exp(xm - m_c), axis=1, keepdims=True)
            m_old = m_acc[...]
            m_new = jnp.maximum(m_old, m_c)
            s_acc[...] = (s_acc[...] * jnp.exp(m_old - m_new)
                          + s_c * jnp.exp(m_c - m_new))
            m_acc[...] = m_new

        @pl.when(j < grid - 1)
        def _full():
            merge(x)

        @pl.when(j == grid - 1)
        def _last():
            merge(jnp.where(loc < v - j * chunk, x, -jnp.inf))
            o_ref[...] = m_acc[...] + jnp.log(s_acc[...])
            g_ref[...] = g_acc[...]

    return pl.pallas_call(
        body,
        grid=(grid,),
        in_specs=[
            pl.BlockSpec((n, chunk), lambda j: (0, j)),
            pl.BlockSpec((n, 1), lambda j: (0, 0)),
        ],
        out_specs=(
            pl.BlockSpec((n, 1), lambda j: (0, 0)),
            pl.BlockSpec((n, 1), lambda j: (0, 0)),
        ),
        out_shape=(
            jax.ShapeDtypeStruct((n, 1), jnp.float32),
            jax.ShapeDtypeStruct((n, 1), jnp.float32),
        ),
        scratch_shapes=[
            pltpu.VMEM((n, 1), jnp.float32),
            pltpu.VMEM((n, 1), jnp.float32),
            pltpu.VMEM((n, 1), jnp.float32),
        ],
    )(x2d, acts)


def _tc_tail(g2, z2, gt, zt, advt, maskt, u):
    b, t = g2.shape
    loc = np.float32(1.0 - _GATE_RATE) * np.float32(b - 1)
    lo = int(np.floor(loc))
    hi = int(np.ceil(loc))
    if lo == hi:
        w_lo, w_hi = np.float32(1.0), np.float32(0.0)
    else:
        w_lo = np.float32(hi) - loc
        w_hi = loc - np.float32(lo)

    def body(g_ref, z_ref, gt_ref, zt_ref, advt_ref, maskt_ref, u_ref,
             alp_ref, gw_ref, gp_ref, dl_ref, price_ref, rate_ref, loss_ref):
        alp_ref[...] = g_ref[...] - z_ref[...]
        alpt = gt_ref[...] - zt_ref[...]
        mt = maskt_ref[...]
        denom = jnp.clip(jnp.sum(mt, axis=0, keepdims=True), 1.0, None)
        mlp = jnp.sum(alpt * mt, axis=0, keepdims=True) / denom
        madv = jnp.sum(advt_ref[...] * mt, axis=0, keepdims=True) / denom
        dl = madv * (-mlp)
        dl_ref[...] = dl
        x = dl * jnp.ones((b, 1), jnp.float32)
        ri = lax.broadcasted_iota(jnp.int32, (b, b), 0)
        ci = lax.broadcasted_iota(jnp.int32, (b, b), 1)
        diag = jnp.where(ri == ci, x, 0.0)
        colv = jnp.sum(diag, axis=1, keepdims=True)
        a = colv * jnp.ones((1, b), jnp.float32)
        before = (x < a) | ((x == a) & (ci < ri))
        rank = jnp.sum(before.astype(jnp.float32), axis=1, keepdims=True)
        s_lo = jnp.sum(jnp.where(rank == lo, colv, 0.0), axis=0, keepdims=True)
        s_hi = jnp.sum(jnp.where(rank == hi, colv, 0.0), axis=0, keepdims=True)
        price = jnp.sum(s_lo * w_lo + s_hi * w_hi, axis=1, keepdims=True)
        price_ref[...] = price
        gl = (dl - price) / jnp.float32(_TEMPERATURE)
        gp = jax.nn.sigmoid(gl)
        gp_ref[...] = gp
        samples = (u_ref[...] < gp).astype(jnp.float32)
        gw = (samples + gp) - gp
        gw_ref[...] = gw
        rate_ref[...] = jnp.sum(samples, axis=1, keepdims=True) / np.float32(b)
        num = -jnp.sum(((gw * advt_ref[...]) * alpt) * mt)
        den = jnp.clip(jnp.sum(mt), 1.0, None)
        loss_ref[...] = jnp.reshape(num / den, (1, 1))

    outs = pl.pallas_call(
        body,
        out_shape=(
            jax.ShapeDtypeStruct((b, t), jnp.float32),
            jax.ShapeDtypeStruct((1, b), jnp.float32),
            jax.ShapeDtypeStruct((1, b), jnp.float32),
            jax.ShapeDtypeStruct((1, b), jnp.float32),
            jax.ShapeDtypeStruct((1, 1), jnp.float32),
            jax.ShapeDtypeStruct((1, 1), jnp.float32),
            jax.ShapeDtypeStruct((1, 1), jnp.float32),
        ),
    )(g2, z2, gt, zt, advt, maskt, u)
    return outs


def kernel(logits, actions, advantages, attention_mask):
    b, t, v = logits.shape
    n = b * t
    x2d = logits.reshape(n, v)
    acts = actions.reshape(n, 1).astype(jnp.int32)

    logz, gathered = _tc_logz_gather(x2d, acts)

    g2 = gathered.reshape(b, t)
    z2 = logz.reshape(b, t)
    mask = attention_mask.astype(jnp.float32)
    u = jax.random.uniform(jax.random.key(42), (b,), jnp.float32).reshape(1, b)

    alp, gw, gp, dl, price, rate, loss = _tc_tail(
        g2, z2, g2.T, z2.T, advantages.astype(jnp.float32).T, mask.T, u)

    return (
        gw.reshape(b),
        gp.reshape(b),
        dl.reshape(b),
        price.reshape(()),
        rate.reshape(()),
        loss.reshape(()),
        alp,
    )
